# baseline (speedup 1.0000x reference)
import functools

import jax
import jax.numpy as jnp
from jax.experimental import pallas as pl
from jax.experimental.pallas import tpu as pltpu


def _router_body(x_ref, wg_ref, cfull_ref, aux_ref, *, nsh):
    x = x_ref[...]
    wg = wg_ref[...]
    logits = jnp.dot(x, wg, preferred_element_type=jnp.float32)
    e_num = logits.shape[1]
    m = jnp.max(logits, axis=1, keepdims=True)
    ex = jnp.exp(logits - m)
    p = ex / jnp.sum(ex, axis=1, keepdims=True)
    lane = jax.lax.broadcasted_iota(jnp.int32, p.shape, 1)
    m1 = jnp.max(p, axis=1, keepdims=True)
    a1 = jnp.min(jnp.where(p == m1, lane, e_num), axis=1, keepdims=True)
    oh1 = lane == a1
    p2 = jnp.where(oh1, -jnp.inf, p)
    m2 = jnp.max(p2, axis=1, keepdims=True)
    a2 = jnp.min(jnp.where(p2 == m2, lane, e_num), axis=1, keepdims=True)
    oh2 = lane == a2
    denom = m1 + m2
    comb = jnp.where(oh1, m1 / denom, 0.0) + jnp.where(oh2, m2 / denom, 0.0)
    ones = jnp.ones((p.shape[0], nsh), dtype=jnp.float32)
    cfull_ref[...] = jnp.concatenate([ones, comb], axis=1)
    aux_ref[0, 0] = jnp.sum(jnp.mean(p, axis=0) ** 2) * e_num


def _moe_body(x_hbm, w1_ref, w2_ref, c_ref, out_ref, x_vmem, sem, *, bt, ntb):
    k = pl.program_id(0)
    hb = pl.program_id(1)
    first = (k == 0) & (hb == 0)

    @pl.when(first)
    def _load_x():
        cp = pltpu.make_async_copy(x_hbm, x_vmem, sem)
        cp.start()
        cp.wait()

    w1 = w1_ref[0]
    w2 = w2_ref[0]
    c = c_ref[...]
    lane = jax.lax.broadcasted_iota(jnp.int32, c.shape, 1)
    scale = jnp.sum(jnp.where(lane == k, c, 0.0), axis=1, keepdims=True)
    for r in range(ntb):
        row = r * bt
        x = x_vmem[pl.ds(row, bt), :].astype(jnp.bfloat16)
        h = jnp.dot(x, w1, preferred_element_type=jnp.float32)
        h = h * jax.nn.sigmoid(h)
        y = jnp.dot(h.astype(jnp.bfloat16), w2, preferred_element_type=jnp.float32)
        y = y * scale[row:row + bt, :]

        @pl.when(first)
        def _init():
            out_ref[pl.ds(row, bt), :] = y

        @pl.when(~first)
        def _acc():
            out_ref[pl.ds(row, bt), :] += y


def kernel(x, Wg, Ws1, Ws2, We1, We2):
    orig_shape = x.shape
    dim = orig_shape[-1]
    xf = x.reshape(-1, dim)
    seq = xf.shape[0]
    e_num = Wg.shape[1]
    nsh = Ws1.shape[0]
    hid = Ws1.shape[2]
    nk = nsh + e_num

    cfull, aux = pl.pallas_call(
        functools.partial(_router_body, nsh=nsh),
        out_shape=[
            jax.ShapeDtypeStruct((seq, nk), jnp.float32),
            jax.ShapeDtypeStruct((1, 1), jnp.float32),
        ],
        out_specs=[
            pl.BlockSpec(memory_space=pltpu.VMEM),
            pl.BlockSpec(memory_space=pltpu.SMEM),
        ],
        compiler_params=pltpu.CompilerParams(
            vmem_limit_bytes=100 * 1024 * 1024,
        ),
    )(xf, Wg)

    w1 = jnp.concatenate([Ws1, We1], axis=0)
    w2 = jnp.concatenate([Ws2, We2], axis=0)

    hc = 384
    hid_pad = ((hid + hc - 1) // hc) * hc
    if hid_pad != hid:
        w1 = jnp.pad(w1, ((0, 0), (0, 0), (0, hid_pad - hid)))
        w2 = jnp.pad(w2, ((0, 0), (0, hid_pad - hid), (0, 0)))
    hid = hid_pad
    w1 = w1.astype(jnp.bfloat16)
    w2 = w2.astype(jnp.bfloat16)

    bt = min(256, seq)
    ntb = seq // bt
    nhb = hid // hc
    out = pl.pallas_call(
        functools.partial(_moe_body, bt=bt, ntb=ntb),
        grid=(nk, nhb),
        in_specs=[
            pl.BlockSpec(memory_space=pl.ANY),
            pl.BlockSpec((1, dim, hc), lambda k, hb: (k, 0, hb)),
            pl.BlockSpec((1, hc, dim), lambda k, hb: (k, hb, 0)),
            pl.BlockSpec((seq, nk), lambda k, hb: (0, 0)),
        ],
        out_specs=pl.BlockSpec((seq, dim), lambda k, hb: (0, 0)),
        out_shape=jax.ShapeDtypeStruct((seq, dim), jnp.float32),
        scratch_shapes=[
            pltpu.VMEM((seq, dim), jnp.float32),
            pltpu.SemaphoreType.DMA,
        ],
        compiler_params=pltpu.CompilerParams(
            dimension_semantics=("arbitrary", "arbitrary"),
            vmem_limit_bytes=100 * 1024 * 1024,
        ),
    )(xf, w1, w2, cfull)

    return out.reshape(orig_shape), aux[0, 0]


# speedup vs baseline: 1.4973x; 1.4973x over previous
import functools

import jax
import jax.numpy as jnp
from jax.experimental import pallas as pl
from jax.experimental.pallas import tpu as pltpu


def _router_body(x_ref, wg_ref, cfull_ref, aux_ref, *, nsh):
    x = x_ref[...]
    wg = wg_ref[...]
    logits = jnp.dot(x, wg, preferred_element_type=jnp.float32)
    e_num = logits.shape[1]
    m = jnp.max(logits, axis=1, keepdims=True)
    ex = jnp.exp(logits - m)
    p = ex / jnp.sum(ex, axis=1, keepdims=True)
    lane = jax.lax.broadcasted_iota(jnp.int32, p.shape, 1)
    m1 = jnp.max(p, axis=1, keepdims=True)
    a1 = jnp.min(jnp.where(p == m1, lane, e_num), axis=1, keepdims=True)
    oh1 = lane == a1
    p2 = jnp.where(oh1, -jnp.inf, p)
    m2 = jnp.max(p2, axis=1, keepdims=True)
    a2 = jnp.min(jnp.where(p2 == m2, lane, e_num), axis=1, keepdims=True)
    oh2 = lane == a2
    denom = m1 + m2
    comb = jnp.where(oh1, m1 / denom, 0.0) + jnp.where(oh2, m2 / denom, 0.0)
    ones = jnp.ones((p.shape[0], nsh), dtype=jnp.float32)
    cfull_ref[...] = jnp.concatenate([ones, comb], axis=1)
    aux_ref[0, 0] = jnp.sum(jnp.mean(p, axis=0) ** 2) * e_num


def _moe_body(x_ref, w1_ref, w2_ref, c_ref, out_ref, *, bt, ntb):
    k = pl.program_id(0)
    w1 = w1_ref[0]
    w2 = w2_ref[0]
    c = c_ref[...]
    lane = jax.lax.broadcasted_iota(jnp.int32, c.shape, 1)
    scale = jnp.sum(jnp.where(lane == k, c, 0.0), axis=1, keepdims=True)
    for r in range(ntb):
        row = r * bt
        x = x_ref[pl.ds(row, bt), :]
        h = jnp.dot(x, w1, preferred_element_type=jnp.float32)
        h = h * jax.nn.sigmoid(h)
        y = jnp.dot(h.astype(jnp.bfloat16), w2, preferred_element_type=jnp.float32)
        y = y * scale[row:row + bt, :]

        @pl.when(k == 0)
        def _init():
            out_ref[pl.ds(row, bt), :] = y

        @pl.when(k > 0)
        def _acc():
            out_ref[pl.ds(row, bt), :] += y


def kernel(x, Wg, Ws1, Ws2, We1, We2):
    orig_shape = x.shape
    dim = orig_shape[-1]
    xf = x.reshape(-1, dim)
    seq = xf.shape[0]
    e_num = Wg.shape[1]
    nsh = Ws1.shape[0]
    hid = Ws1.shape[2]
    nk = nsh + e_num

    cfull, aux = pl.pallas_call(
        functools.partial(_router_body, nsh=nsh),
        out_shape=[
            jax.ShapeDtypeStruct((seq, nk), jnp.float32),
            jax.ShapeDtypeStruct((1, 1), jnp.float32),
        ],
        out_specs=[
            pl.BlockSpec(memory_space=pltpu.VMEM),
            pl.BlockSpec(memory_space=pltpu.SMEM),
        ],
        compiler_params=pltpu.CompilerParams(
            vmem_limit_bytes=100 * 1024 * 1024,
        ),
    )(xf, Wg)

    w1 = jnp.concatenate([Ws1, We1], axis=0)
    w2 = jnp.concatenate([Ws2, We2], axis=0)

    hc = 384
    hid_pad = ((hid + hc - 1) // hc) * hc
    if hid_pad != hid:
        w1 = jnp.pad(w1, ((0, 0), (0, 0), (0, hid_pad - hid)))
        w2 = jnp.pad(w2, ((0, 0), (0, hid_pad - hid), (0, 0)))
    hid = hid_pad
    w1 = w1.astype(jnp.bfloat16)
    w2 = w2.astype(jnp.bfloat16)

    bt = min(256, seq)
    ntb = seq // bt
    xbf = xf.astype(jnp.bfloat16)
    out = pl.pallas_call(
        functools.partial(_moe_body, bt=bt, ntb=ntb),
        grid=(nk,),
        in_specs=[
            pl.BlockSpec((seq, dim), lambda k: (0, 0)),
            pl.BlockSpec((1, dim, hid), lambda k: (k, 0, 0)),
            pl.BlockSpec((1, hid, dim), lambda k: (k, 0, 0)),
            pl.BlockSpec((seq, nk), lambda k: (0, 0)),
        ],
        out_specs=pl.BlockSpec((seq, dim), lambda k: (0, 0)),
        out_shape=jax.ShapeDtypeStruct((seq, dim), jnp.float32),
        compiler_params=pltpu.CompilerParams(
            dimension_semantics=("arbitrary",),
            vmem_limit_bytes=100 * 1024 * 1024,
        ),
    )(xbf, w1, w2, cfull)

    return out.reshape(orig_shape), aux[0, 0]
